# baseline (speedup 1.0000x reference)
import functools

import jax
import jax.numpy as jnp
from jax import lax
from jax.experimental import pallas as pl
from jax.experimental.pallas import tpu as pltpu
from jax.experimental.pallas import tpu_sc as plsc

N = 50000
E = 800000
T = 800000
H = 64
MAXN = 4
NL = 16
NLAYERS = 4
MAXZ = 94
CUTOFF = 5.0
TB_CUTOFF = 4.0

NC = 2
NS = 16
NW = NC * NS

CH = 1000
CHP = 1008


def _swish(x):
    return x * jax.nn.sigmoid(x)


def _poly_cutoff(r, c):
    t = r / c
    f = 1.0 - 6.0 * t**5 + 15.0 * t**4 - 10.0 * t**3
    return jnp.where(r < c, f, 0.0)


@functools.partial(jax.jit, static_argnames=("d",))
def _sc_gather(table, idx, d):
    b = idx.shape[0]
    total_chunks = -(-b // CH)
    nchw = -(-total_chunks // NW)
    mesh = plsc.VectorSubcoreMesh(core_axis_name="c", subcore_axis_name="s")

    @functools.partial(
        pl.kernel,
        out_type=jax.ShapeDtypeStruct((b, d), table.dtype),
        mesh=mesh,
        compiler_params=pltpu.CompilerParams(use_tc_tiling_on_sc=False),
        scratch_types=[
            pltpu.VMEM((CHP,), jnp.int32),
            pltpu.VMEM((CHP, d), table.dtype),
            pltpu.SemaphoreType.DMA,
        ],
    )
    def k(table_hbm, idx_hbm, out_hbm, idx_v, rows_v, sem):
        wid = lax.axis_index("s") * NC + lax.axis_index("c")
        idx_v[pl.ds(CHP - 16, 16)] = jnp.zeros((16,), jnp.int32)

        def body(i, carry):
            start = jnp.minimum((wid * nchw + i) * CH, b - CH)
            pltpu.sync_copy(idx_hbm.at[pl.ds(start, CH)], idx_v.at[pl.ds(0, CH)])
            pltpu.async_copy(table_hbm.at[idx_v], rows_v, sem).wait()
            pltpu.sync_copy(rows_v.at[pl.ds(0, CH)], out_hbm.at[pl.ds(start, CH)])
            return carry

        lax.fori_loop(0, nchw, body, 0)

    return k(table, idx)


SCH = 200
SCHP = 208


@functools.partial(jax.jit, static_argnames=("s_rows", "d", "schunk", "phases"))
def _sc_segsum(vals, idx, init, zrows, s_rows, d, schunk, phases):
    b = vals.shape[0]
    per_tile = b // NS
    nch = per_tile // SCH
    rpt = schunk // NS
    zr = zrows.shape[0]
    nz = rpt // zr
    mesh = plsc.VectorSubcoreMesh(core_axis_name="c", subcore_axis_name="s")
    have_init = init is not None
    in_args = (vals, idx) + ((init,) if have_init else (zrows,))

    @functools.partial(
        pl.kernel,
        out_type=jax.ShapeDtypeStruct((s_rows, d), jnp.float32),
        mesh=mesh,
        compiler_params=pltpu.CompilerParams(use_tc_tiling_on_sc=False),
        scratch_types=[
            pltpu.VMEM((SCHP,), jnp.int32),
            pltpu.VMEM((SCHP,), jnp.int32),
            pltpu.VMEM((SCHP, d), jnp.float32),
            pltpu.VMEM_SHARED((schunk + 16, d), jnp.float32),
            pltpu.SemaphoreType.DMA,
        ],
    )
    def k(vals_hbm, idx_hbm, src_hbm, out_hbm, idx_v, idx2_v, vals_v, acc, sem):
        c = lax.axis_index("c")
        s = lax.axis_index("s")
        idx_v[pl.ds(SCHP - 16, 16)] = jnp.full((16,), -1, jnp.int32)

        for p in range(phases):
            gb = (p * NC + c) * schunk
            if have_init:
                start = jnp.minimum(gb + s * rpt, s_rows - rpt)
                pltpu.sync_copy(src_hbm.at[pl.ds(start, rpt)],
                                acc.at[pl.ds(start - gb, rpt)])
            else:
                for z in range(nz):
                    pltpu.sync_copy(src_hbm,
                                    acc.at[pl.ds(s * rpt + z * zr, zr)])
            plsc.subcore_barrier()

            def body(i, carry):
                start = s * per_tile + i * SCH
                pltpu.sync_copy(idx_hbm.at[pl.ds(start, SCH)],
                                idx_v.at[pl.ds(0, SCH)])
                pltpu.sync_copy(vals_hbm.at[pl.ds(start, SCH)],
                                vals_v.at[pl.ds(0, SCH)])
                for g in range(SCHP // 16):
                    v = idx_v[pl.ds(g * 16, 16)]
                    loc = v - gb
                    ok = (loc >= 0) & (loc < schunk)
                    idx2_v[pl.ds(g * 16, 16)] = jnp.where(ok, loc, schunk)
                pltpu.sync_copy(vals_v, acc.at[idx2_v], add=True)
                return carry

            lax.fori_loop(0, nch, body, 0)
            plsc.subcore_barrier()

            @pl.when(gb < s_rows)
            def _():
                start = jnp.minimum(gb + s * rpt, s_rows - rpt)
                pltpu.sync_copy(acc.at[pl.ds(start - gb, rpt)],
                                out_hbm.at[pl.ds(start, rpt)])
            plsc.subcore_barrier()

    return k(*in_args)


def _chain(tok, arr):
    arr2, _ = lax.optimization_barrier((arr, tok))
    return arr2


def _grid_call(body, nouts, out_cols, rows, rblk, ins, in_cols):
    grid = rows // rblk
    in_specs = []
    for a, c in zip(ins, in_cols):
        if c is None:
            in_specs.append(pl.BlockSpec(a.shape, lambda i, nd=a.ndim: (0,) * nd))
        else:
            in_specs.append(pl.BlockSpec((rblk, c), lambda i: (i, 0)))
    out_specs = [pl.BlockSpec((rblk, c), lambda i: (i, 0)) for c in out_cols]
    out_shape = [jax.ShapeDtypeStruct((rows, c), jnp.float32) for c in out_cols]
    if nouts == 1:
        out_specs, out_shape = out_specs[0], out_shape[0]
    return pl.pallas_call(
        body, grid=(grid,), in_specs=in_specs,
        out_specs=out_specs, out_shape=out_shape,
    )(*ins)


def _edge_init_body(dist_ref, wenc_ref, rbf_ref, e0_ref):
    d = dist_ref[...]
    narr = jnp.arange(1, MAXN + 1).astype(jnp.float32)
    rbf = jnp.sqrt(2.0 / CUTOFF) * jnp.sin(narr[None, :] * jnp.pi * d / CUTOFF) / d
    rbf_ref[...] = rbf
    e0_ref[...] = _swish(jnp.dot(rbf, wenc_ref[...],
                                 preferred_element_type=jnp.float32))


def _triple_basis_body(vd0_ref, vd1_ref, out_ref):
    vd0 = vd0_ref[...]
    vd1 = vd1_ref[...]
    rij = vd0[:, 3:4]
    rik = vd1[:, 3:4]
    dot = jnp.sum(vd0[:, :3] * vd1[:, :3], axis=1, keepdims=True)
    ct = jnp.clip(dot / (rij * rik), -1.0 + 1e-07, 1.0 - 1e-07)
    narr = jnp.arange(1, MAXN + 1).astype(jnp.float32)
    sb = jnp.sin(narr[None, :] * jnp.pi * rik / TB_CUTOFF) / rik
    fc = _poly_cutoff(rij, TB_CUTOFF) * _poly_cutoff(rik, TB_CUTOFF)
    p0 = jnp.ones_like(ct)
    p1 = ct
    p2 = 0.5 * (3.0 * ct * ct - 1.0)
    p3 = 0.5 * (5.0 * ct * ct * ct - 3.0 * ct)
    sbf = sb * fc
    out_ref[...] = jnp.concatenate(
        [p0 * sbf, p1 * sbf, p2 * sbf, p3 * sbf], axis=1)


def _gate_body(x_ref, wa_ref, ba_ref, out_ref):
    out_ref[...] = jax.nn.sigmoid(
        jnp.dot(x_ref[...], wa_ref[...], preferred_element_type=jnp.float32)
        + ba_ref[...])


def _mul_body(a_ref, b_ref, out_ref):
    out_ref[...] = a_ref[...] * b_ref[...]


def _layer_body(esum_ref, edge_ref, as_ref, at_ref, rbf_ref,
                w3m_ref, b3m_ref, w3g_ref, b3g_ref,
                we_ref, be_ref, weg_ref, beg_ref, wez_ref,
                wn_ref, bn_ref, wng_ref, bng_ref, wnz_ref,
                eout_ref, msg_ref):
    f32 = jnp.float32
    esum = esum_ref[...]
    a_s = as_ref[...]
    a_t = at_ref[...]
    rbf = rbf_ref[...]
    m3 = jnp.dot(esum, w3m_ref[...], preferred_element_type=f32) + b3m_ref[...]
    g3 = jnp.dot(esum, w3g_ref[...], preferred_element_type=f32) + b3g_ref[...]
    e1 = edge_ref[...] + _swish(m3) * jax.nn.sigmoid(g3)

    we = we_ref[...]
    weg = weg_ref[...]
    m = (jnp.dot(a_s, we[0:H], preferred_element_type=f32)
         + jnp.dot(a_t, we[H:2 * H], preferred_element_type=f32)
         + jnp.dot(e1, we[2 * H:3 * H], preferred_element_type=f32)
         + be_ref[...])
    g = (jnp.dot(a_s, weg[0:H], preferred_element_type=f32)
         + jnp.dot(a_t, weg[H:2 * H], preferred_element_type=f32)
         + jnp.dot(e1, weg[2 * H:3 * H], preferred_element_type=f32)
         + beg_ref[...])
    rz = jnp.dot(rbf, wez_ref[...], preferred_element_type=f32)
    e2 = e1 + _swish(m) * jax.nn.sigmoid(g) * rz
    eout_ref[...] = e2

    wn = wn_ref[...]
    wng = wng_ref[...]
    m2 = (jnp.dot(a_s, wn[0:H], preferred_element_type=f32)
          + jnp.dot(a_t, wn[H:2 * H], preferred_element_type=f32)
          + jnp.dot(e2, wn[2 * H:3 * H], preferred_element_type=f32)
          + bn_ref[...])
    g2 = (jnp.dot(a_s, wng[0:H], preferred_element_type=f32)
          + jnp.dot(a_t, wng[H:2 * H], preferred_element_type=f32)
          + jnp.dot(e2, wng[2 * H:3 * H], preferred_element_type=f32)
          + bng_ref[...])
    rz2 = jnp.dot(rbf, wnz_ref[...], preferred_element_type=f32)
    msg_ref[...] = _swish(m2) * jax.nn.sigmoid(g2) * rz2


def _head_body(x_ref, wm_ref, bm_ref, wg_ref, bg_ref, out_ref):
    f32 = jnp.float32
    m = x_ref[...]
    g = m
    wm = wm_ref[...]
    bm = bm_ref[...]
    wg = wg_ref[...]
    bg = bg_ref[...]
    for i in range(3):
        m = jnp.dot(m, wm[i], preferred_element_type=f32) + bm[i:i + 1, :]
        g = jnp.dot(g, wg[i], preferred_element_type=f32) + bg[i:i + 1, :]
        if i < 2:
            m = _swish(m)
            g = _swish(g)
    out_ref[...] = m * jax.nn.sigmoid(g)


RB_E = 1600
RB_N = 2000

SCHUNK_N = 16 * 1568
SCHUNK_E = 16 * 7000
PHASES_E = 4


def kernel(atomic_numbers, a2a_edge_index, a2a_distance, a2a_vector,
           a2ee2a_edge_index, a2ee2a_distance, params):
    del a2ee2a_distance
    p = params
    idx_s = a2a_edge_index[0].astype(jnp.int32)
    idx_t = a2a_edge_index[1].astype(jnp.int32)
    tb0 = a2ee2a_edge_index[0].astype(jnp.int32)
    tb1 = a2ee2a_edge_index[1].astype(jnp.int32)
    dist = a2a_distance.reshape(E, 1)
    table8 = jnp.concatenate(
        [a2a_vector, dist,
         idx_s.astype(jnp.float32).reshape(E, 1),
         jnp.zeros((E, 3), jnp.float32)], axis=1)

    zrows_e = jnp.zeros((SCHUNK_E // NS, NL), jnp.float32)
    zrows_n = jnp.zeros((8, H), jnp.float32)

    rbf, edge_attr = _grid_call(_edge_init_body, 2, (MAXN, H), E, RB_E,
                                (dist, p['Wenc']), (1, None))
    vd0 = _sc_gather(table8, tb0, 8)
    vd1 = _sc_gather(table8, _chain(vd0, tb1), 8)
    ci = vd1[:, 4].astype(jnp.int32)
    tbfc = _grid_call(_triple_basis_body, 1, (NL,), T, RB_E,
                      (vd0, vd1), (8, 8))
    atom_attr = _sc_gather(p['Wemb'],
                           _chain(vd1, atomic_numbers.astype(jnp.int32)), H)

    for l in range(NLAYERS):
        gate = _grid_call(_gate_body, 1, (NL,), N, RB_N,
                          (atom_attr, p['Wa'][l], p['ba'][l].reshape(1, NL)),
                          (H, None, None))
        gate_t = _sc_gather(gate, _chain(atom_attr, ci), NL)
        weighted = _grid_call(_mul_body, 1, (NL,), T, RB_E,
                              (tbfc, gate_t), (NL, NL))
        esum = _sc_segsum(weighted, tb0, None, zrows_e, E, NL,
                          SCHUNK_E, PHASES_E)
        a_s = _sc_gather(atom_attr, _chain(esum, idx_s), H)
        a_t = _sc_gather(atom_attr, _chain(a_s, idx_t), H)
        edge_attr, msg = _grid_call(
            _layer_body, 2, (H, H), E, RB_E,
            (esum, edge_attr, a_s, a_t, rbf,
             p['W3m'][l], p['b3m'][l].reshape(1, H),
             p['W3g'][l], p['b3g'][l].reshape(1, H),
             p['We'][l], p['be'][l].reshape(1, H),
             p['Weg'][l], p['beg'][l].reshape(1, H), p['Wez'][l],
             p['Wn'][l], p['bn'][l].reshape(1, H),
             p['Wng'][l], p['bng'][l].reshape(1, H), p['Wnz'][l]),
            (NL, H, H, H, MAXN) + (None,) * 14)
        atom_attr = _sc_segsum(msg, idx_t, atom_attr, zrows_n, N, H,
                               SCHUNK_N, 1)

    x_e = _grid_call(_head_body, 1, (H,), N, RB_N,
                     (atom_attr, p['WEm'], p['bEm'], p['WEg'], p['bEg']),
                     (H, None, None, None, None))
    x_f = _grid_call(_head_body, 1, (H,), E, RB_E,
                     (edge_attr, p['WFm'], p['bFm'], p['WFg'], p['bFg']),
                     (H, None, None, None, None))
    return (atom_attr, edge_attr, x_e, x_f)
